# baseline (speedup 1.0000x reference)
import functools

import jax
import jax.numpy as jnp
from jax import lax
from jax.experimental import pallas as pl
from jax.experimental.pallas import tpu as pltpu
from jax.experimental.pallas import tpu_sc as plsc

_N = 8192
_K = 4
_E = _N * _K

_SC_CORES = 2
_SC_SUBCORES = 16
_SC_WORKERS = _SC_CORES * _SC_SUBCORES

_KNN_ROWS = 256
_BE = 4096
_BN = 4096
_GW = 128
_GCHUNK = 512


def _knn_body(pc_blk_ref, pcT_ref, idx_ref):
    pcb = pc_blk_ref[...]
    pcT = pcT_ref[...]
    r = pcb.shape[0]
    sq_c = jnp.sum(pcT * pcT, axis=0, keepdims=True)
    sq_r = jnp.sum(pcb * pcb, axis=1, keepdims=True)
    dot = jnp.dot(pcb.astype(jnp.bfloat16), pcT.astype(jnp.bfloat16),
                  preferred_element_type=jnp.float32)
    d = sq_r + sq_c - 2.0 * dot
    lane = lax.broadcasted_iota(jnp.int32, (r, 128), 1)
    big_i = jnp.int32(2**30)
    inf = jnp.float32(jnp.inf)
    v0 = d[:, 0:128]
    i0 = jnp.zeros((r, 128), jnp.int32)
    v1 = jnp.full((r, 128), inf)
    i1 = jnp.zeros((r, 128), jnp.int32)
    v2 = jnp.full((r, 128), inf)
    i2 = jnp.zeros((r, 128), jnp.int32)
    for c in range(1, _N // 128):
        dc = d[:, c * 128:(c + 1) * 128]
        cc = jnp.int32(c)
        c0 = dc < v0
        c1 = dc < v1
        c2 = dc < v2
        v2 = jnp.where(c1, v1, jnp.where(c2, dc, v2))
        i2 = jnp.where(c1, i1, jnp.where(c2, cc, i2))
        v1 = jnp.where(c0, v0, jnp.where(c1, dc, v1))
        i1 = jnp.where(c0, i0, jnp.where(c1, cc, i1))
        v0 = jnp.where(c0, dc, v0)
        i0 = jnp.where(c0, cc, i0)
    w = jnp.concatenate([v0, v1, v2], axis=1)
    jf = jnp.concatenate(
        [i0 * 128 + lane, i1 * 128 + lane, i2 * 128 + lane], axis=1)
    cols = []
    for t in range(_K):
        m = jnp.min(w, axis=1, keepdims=True)
        j = jnp.min(jnp.where(w == m, jf, big_i), axis=1, keepdims=True)
        cols.append(j)
        if t < _K - 1:
            w = jnp.where(jf == j, inf, w)
    idx_ref[...] = jnp.concatenate(cols, axis=1)


def _knn(pc2d):
    pcT = pc2d.T
    return pl.pallas_call(
        _knn_body,
        grid=(_N // _KNN_ROWS,),
        in_specs=[
            pl.BlockSpec((_KNN_ROWS, 3), lambda i: (i, 0)),
            pl.BlockSpec((3, _N), lambda i: (0, 0)),
        ],
        out_specs=pl.BlockSpec((_KNN_ROWS, _K), lambda i: (i, 0)),
        out_shape=jax.ShapeDtypeStruct((_N, _K), jnp.int32),
    )(pc2d, pcT)


def _sc_gather(table, idx):
    b_per_w = _E // _SC_WORKERS
    n_chunks = b_per_w // _GCHUNK
    mesh = plsc.VectorSubcoreMesh(core_axis_name="c", subcore_axis_name="s")

    @functools.partial(
        pl.kernel,
        mesh=mesh,
        out_type=jax.ShapeDtypeStruct((_E, _GW), jnp.float32),
        scratch_types=[
            pltpu.VMEM((b_per_w,), jnp.int32),
            pltpu.VMEM((_GCHUNK, _GW), jnp.float32),
            pltpu.SemaphoreType.DMA,
        ],
    )
    def k(table_hbm, idx_hbm, out_hbm, idx_v, rows_v, sem):
        wid = lax.axis_index("s") * _SC_CORES + lax.axis_index("c")
        base = wid * b_per_w
        pltpu.sync_copy(idx_hbm.at[pl.ds(base, b_per_w)], idx_v)
        for c in range(n_chunks):
            pltpu.async_copy(
                table_hbm.at[idx_v.at[pl.ds(c * _GCHUNK, _GCHUNK)]],
                rows_v, sem).wait()
            pltpu.sync_copy(rows_v,
                            out_hbm.at[pl.ds(base + c * _GCHUNK, _GCHUNK)])

    return k(table, idx)


def _accum_stats(s_ref, h, step):
    c = h.shape[1]
    part = jnp.concatenate(
        [jnp.sum(h, axis=0, keepdims=True),
         jnp.sum(h * h, axis=0, keepdims=True),
         jnp.zeros((6, c), jnp.float32)], axis=0)

    @pl.when(step == 0)
    def _():
        s_ref[...] = jnp.zeros_like(s_ref)

    s_ref[...] = s_ref[...] + part


def _norm_lrelu(h, s, pack, c):
    mean = s[0:1, :] * (1.0 / _E)
    var = s[1:2, :] * (1.0 / _E) - mean * mean
    g = pack[:, c:2 * c]
    be = pack[:, 2 * c:3 * c]
    h = (h - mean) / jnp.sqrt(var + 1e-5) * g + be
    return jnp.where(h >= 0, h, 0.1 * h)


def _l1p1_body(g0_ref, pc_ref, w_ref, p_ref, h_ref, ef_ref, s_ref):
    g0 = g0_ref[...]
    pc = pc_ref[...]
    ef = g0[:, 3:6] - pc
    x = jnp.concatenate([g0[:, 0:3], ef], axis=1)
    c = w_ref.shape[1]
    h = jnp.dot(x, w_ref[...], preferred_element_type=jnp.float32) \
        + p_ref[...][:, 0:c]
    h_ref[...] = h
    ef_ref[...] = jnp.concatenate([ef, jnp.zeros((ef.shape[0], 1), jnp.float32)],
                                  axis=1)
    _accum_stats(s_ref, h, pl.program_id(0))


def _lnp1_body(g_ref, ef_ref, w_ref, p_ref, h_ref, s_ref, *, cin):
    x = jnp.concatenate([g_ref[...][:, 0:cin], ef_ref[...][:, 0:3]], axis=1)
    c = w_ref.shape[1]
    h = jnp.dot(x, w_ref[...], preferred_element_type=jnp.float32) \
        + p_ref[...][:, 0:c]
    h_ref[...] = h
    _accum_stats(s_ref, h, pl.program_id(0))


def _mid_body(hin_ref, s_in_ref, pprev_ref, w_ref, p_ref, h_ref, s_ref, *, c):
    h = _norm_lrelu(hin_ref[...], s_in_ref[...], pprev_ref[...], c)
    h = jnp.dot(h, w_ref[...], preferred_element_type=jnp.float32) \
        + p_ref[...][:, 0:c]
    h_ref[...] = h
    _accum_stats(s_ref, h, pl.program_id(0))


def _pool_body(h0_ref, h1_ref, h2_ref, h3_ref, s_ref, p_ref, out_ref, *, c):
    s = s_ref[...]
    p = p_ref[...]
    a = jnp.maximum(_norm_lrelu(h0_ref[...], s, p, c),
                    _norm_lrelu(h1_ref[...], s, p, c))
    b = jnp.maximum(_norm_lrelu(h2_ref[...], s, p, c),
                    _norm_lrelu(h3_ref[...], s, p, c))
    m = jnp.maximum(a, b)
    out_ref[...] = jnp.concatenate(
        [m, jnp.zeros((m.shape[0], _GW - c), jnp.float32)], axis=1)


def _final_body(h0_ref, h1_ref, h2_ref, h3_ref, s_ref, p_ref, fcw_ref,
                fcb_ref, flow_ref, out_ref):
    c = 64
    s = s_ref[...]
    p = p_ref[...]
    a = jnp.maximum(_norm_lrelu(h0_ref[...], s, p, c),
                    _norm_lrelu(h1_ref[...], s, p, c))
    b = jnp.maximum(_norm_lrelu(h2_ref[...], s, p, c),
                    _norm_lrelu(h3_ref[...], s, p, c))
    m = jnp.maximum(a, b)
    y = jnp.dot(m, fcw_ref[...], preferred_element_type=jnp.float32) \
        + fcb_ref[...]
    out_ref[...] = flow_ref[...] + y


def _pack_layer_params(p, w):
    def pack(i):
        return jnp.concatenate(
            [p['fc%d_b' % i], p['bn%d_g' % i], p['bn%d_b' % i]]
        ).reshape(1, 3 * w)
    return (p['fc1_w'].T, pack(1), p['fc2_w'].T, pack(2), p['fc3_w'].T, pack(3))


def _edge_specs(c):
    blocks_per_n = _N // _BN
    return [
        pl.BlockSpec((_BN, c), functools.partial(
            lambda k, i: (k * blocks_per_n + i, 0), k))
        for k in range(_K)
    ]


def _stats_spec(c):
    return pl.BlockSpec((8, c), lambda i: (0, 0))


def _full(shape):
    return pl.BlockSpec(shape, lambda i: tuple(0 for _ in shape))


def _setconv(gathered, ef, pc2d, params, cin, c, first):
    wt1, p1, wt2, p2, wt3, p3 = _pack_layer_params(params, c)
    grid_e = (_E // _BE,)
    h_shape = jax.ShapeDtypeStruct((_E, c), jnp.float32)
    s_shape = jax.ShapeDtypeStruct((8, c), jnp.float32)

    if first:
        h1, ef, s1 = pl.pallas_call(
            _l1p1_body,
            grid=grid_e,
            in_specs=[
                pl.BlockSpec((_BE, _GW), lambda i: (i, 0)),
                pl.BlockSpec((_BE, 3),
                             lambda i: (i % (_N // _BE), 0)),
                _full(wt1.shape), _full(p1.shape),
            ],
            out_specs=[pl.BlockSpec((_BE, c), lambda i: (i, 0)),
                       pl.BlockSpec((_BE, 4), lambda i: (i, 0)),
                       _stats_spec(c)],
            out_shape=(h_shape, jax.ShapeDtypeStruct((_E, 4), jnp.float32),
                       s_shape),
        )(gathered, pc2d, wt1, p1)
    else:
        h1, s1 = pl.pallas_call(
            functools.partial(_lnp1_body, cin=cin),
            grid=grid_e,
            in_specs=[
                pl.BlockSpec((_BE, _GW), lambda i: (i, 0)),
                pl.BlockSpec((_BE, 4), lambda i: (i, 0)),
                _full(wt1.shape), _full(p1.shape),
            ],
            out_specs=[pl.BlockSpec((_BE, c), lambda i: (i, 0)),
                       _stats_spec(c)],
            out_shape=(h_shape, s_shape),
        )(gathered, ef, wt1, p1)

    def mid(hin, s_in, pprev, wt, p):
        return pl.pallas_call(
            functools.partial(_mid_body, c=c),
            grid=grid_e,
            in_specs=[
                pl.BlockSpec((_BE, c), lambda i: (i, 0)),
                _stats_spec(c), _full(pprev.shape),
                _full(wt.shape), _full(p.shape),
            ],
            out_specs=[pl.BlockSpec((_BE, c), lambda i: (i, 0)),
                       _stats_spec(c)],
            out_shape=(h_shape, s_shape),
        )(hin, s_in, pprev, wt, p)

    h2, s2 = mid(h1, s1, p1, wt2, p2)
    h3, s3 = mid(h2, s2, p2, wt3, p3)
    return h3, s3, p3, ef


def kernel(pc, flow, params):
    pc2d = pc.reshape(_N, 3)
    flow2d = flow.reshape(_N, 3)

    idx = _knn(pc2d)
    edges = idx.T.reshape(_E)

    grid_n = (_N // _BN,)

    table0 = jnp.concatenate(
        [flow2d, pc2d, jnp.zeros((_N, _GW - 6), jnp.float32)], axis=1)
    g0 = _sc_gather(table0, edges)
    h3, s3, p3, ef = _setconv(g0, None, pc2d, params['sc1'], 3, 16, True)
    x1 = pl.pallas_call(
        functools.partial(_pool_body, c=16),
        grid=grid_n,
        in_specs=_edge_specs(16) + [_stats_spec(16), _full(p3.shape)],
        out_specs=pl.BlockSpec((_BN, _GW), lambda i: (i, 0)),
        out_shape=jax.ShapeDtypeStruct((_N, _GW), jnp.float32),
    )(h3, h3, h3, h3, s3, p3)

    g1 = _sc_gather(x1, edges)
    h3, s3, p3, _ = _setconv(g1, ef, pc2d, params['sc2'], 16, 32, False)
    x2 = pl.pallas_call(
        functools.partial(_pool_body, c=32),
        grid=grid_n,
        in_specs=_edge_specs(32) + [_stats_spec(32), _full(p3.shape)],
        out_specs=pl.BlockSpec((_BN, _GW), lambda i: (i, 0)),
        out_shape=jax.ShapeDtypeStruct((_N, _GW), jnp.float32),
    )(h3, h3, h3, h3, s3, p3)

    g2 = _sc_gather(x2, edges)
    h3, s3, p3, _ = _setconv(g2, ef, pc2d, params['sc3'], 32, 64, False)
    fcw = params['fc_w'].T
    fcb = params['fc_b'].reshape(1, 3)
    out = pl.pallas_call(
        _final_body,
        grid=grid_n,
        in_specs=_edge_specs(64) + [
            _stats_spec(64), _full(p3.shape), _full(fcw.shape),
            _full(fcb.shape),
            pl.BlockSpec((_BN, 3), lambda i: (i, 0)),
        ],
        out_specs=pl.BlockSpec((_BN, 3), lambda i: (i, 0)),
        out_shape=jax.ShapeDtypeStruct((_N, 3), jnp.float32),
    )(h3, h3, h3, h3, s3, p3, fcw, fcb, flow2d)

    return out.reshape(1, _N, 3)


# speedup vs baseline: 1.0373x; 1.0373x over previous
import functools

import jax
import jax.numpy as jnp
from jax import lax
from jax.experimental import pallas as pl
from jax.experimental.pallas import tpu as pltpu
from jax.experimental.pallas import tpu_sc as plsc

_N = 8192
_K = 4
_E = _N * _K

_SC_CORES = 2
_SC_SUBCORES = 16
_SC_WORKERS = _SC_CORES * _SC_SUBCORES

_KNN_ROWS = 512
_BE = 4096
_BN = 4096
_GW = 128
_GCHUNK = 512


def _knn_body(pc_blk_ref, pcT_ref, idx_ref):
    pcb = pc_blk_ref[...]
    pcT = pcT_ref[...]
    r = pcb.shape[0]
    sq_c = jnp.sum(pcT * pcT, axis=0, keepdims=True)
    sq_r = jnp.sum(pcb * pcb, axis=1, keepdims=True)
    dot = jnp.dot(pcb.astype(jnp.bfloat16), pcT.astype(jnp.bfloat16),
                  preferred_element_type=jnp.float32)
    d = sq_r + sq_c - 2.0 * dot
    lane = lax.broadcasted_iota(jnp.int32, (r, 128), 1)
    big_i = jnp.int32(2**30)
    inf = jnp.float32(jnp.inf)
    v0 = d[:, 0:128]
    i0 = jnp.zeros((r, 128), jnp.int32)
    v1 = jnp.full((r, 128), inf)
    i1 = jnp.zeros((r, 128), jnp.int32)
    v2 = jnp.full((r, 128), inf)
    i2 = jnp.zeros((r, 128), jnp.int32)
    for c in range(1, _N // 128):
        dc = d[:, c * 128:(c + 1) * 128]
        cc = jnp.int32(c)
        c0 = dc < v0
        c1 = dc < v1
        c2 = dc < v2
        v2 = jnp.where(c1, v1, jnp.where(c2, dc, v2))
        i2 = jnp.where(c1, i1, jnp.where(c2, cc, i2))
        v1 = jnp.where(c0, v0, jnp.where(c1, dc, v1))
        i1 = jnp.where(c0, i0, jnp.where(c1, cc, i1))
        v0 = jnp.where(c0, dc, v0)
        i0 = jnp.where(c0, cc, i0)
    w = jnp.concatenate([v0, v1, v2], axis=1)
    jf = jnp.concatenate(
        [i0 * 128 + lane, i1 * 128 + lane, i2 * 128 + lane], axis=1)
    cols = []
    for t in range(_K):
        m = jnp.min(w, axis=1, keepdims=True)
        j = jnp.min(jnp.where(w == m, jf, big_i), axis=1, keepdims=True)
        cols.append(j)
        if t < _K - 1:
            w = jnp.where(jf == j, inf, w)
    idx_ref[...] = jnp.concatenate(cols, axis=1)


def _knn(pc2d):
    pcT = pc2d.T
    return pl.pallas_call(
        _knn_body,
        grid=(_N // _KNN_ROWS,),
        in_specs=[
            pl.BlockSpec((_KNN_ROWS, 3), lambda i: (i, 0)),
            pl.BlockSpec((3, _N), lambda i: (0, 0)),
        ],
        out_specs=pl.BlockSpec((_KNN_ROWS, _K), lambda i: (i, 0)),
        out_shape=jax.ShapeDtypeStruct((_N, _K), jnp.int32),
    )(pc2d, pcT)


def _sc_gather(table, idx):
    b_per_w = _E // _SC_WORKERS
    n_chunks = b_per_w // _GCHUNK
    mesh = plsc.VectorSubcoreMesh(core_axis_name="c", subcore_axis_name="s")

    @functools.partial(
        pl.kernel,
        mesh=mesh,
        out_type=jax.ShapeDtypeStruct((_E, _GW), jnp.float32),
        scratch_types=[
            pltpu.VMEM((b_per_w,), jnp.int32),
            pltpu.VMEM((_GCHUNK, _GW), jnp.float32),
            pltpu.SemaphoreType.DMA,
        ],
    )
    def k(table_hbm, idx_hbm, out_hbm, idx_v, rows_v, sem):
        wid = lax.axis_index("s") * _SC_CORES + lax.axis_index("c")
        base = wid * b_per_w
        pltpu.sync_copy(idx_hbm.at[pl.ds(base, b_per_w)], idx_v)
        for c in range(n_chunks):
            pltpu.async_copy(
                table_hbm.at[idx_v.at[pl.ds(c * _GCHUNK, _GCHUNK)]],
                rows_v, sem).wait()
            pltpu.sync_copy(rows_v,
                            out_hbm.at[pl.ds(base + c * _GCHUNK, _GCHUNK)])

    return k(table, idx)


def _accum_stats(s_ref, h, step):
    c = h.shape[1]
    part = jnp.concatenate(
        [jnp.sum(h, axis=0, keepdims=True),
         jnp.sum(h * h, axis=0, keepdims=True),
         jnp.zeros((6, c), jnp.float32)], axis=0)

    @pl.when(step == 0)
    def _():
        s_ref[...] = jnp.zeros_like(s_ref)

    s_ref[...] = s_ref[...] + part


def _bdot(x, w):
    return jnp.dot(x.astype(jnp.bfloat16), w.astype(jnp.bfloat16),
                   preferred_element_type=jnp.float32)


def _norm_lrelu(h, s, pack, c):
    mean = s[0:1, :] * (1.0 / _E)
    var = s[1:2, :] * (1.0 / _E) - mean * mean
    g = pack[:, c:2 * c]
    be = pack[:, 2 * c:3 * c]
    h = (h - mean) / jnp.sqrt(var + 1e-5) * g + be
    return jnp.where(h >= 0, h, 0.1 * h)


def _l1p1_body(g0_ref, pc_ref, w_ref, p_ref, h_ref, ef_ref, s_ref):
    g0 = g0_ref[...]
    pc = pc_ref[...]
    ef = g0[:, 3:6] - pc
    x = jnp.concatenate([g0[:, 0:3], ef], axis=1)
    c = w_ref.shape[1]
    h = _bdot(x, w_ref[...]) + p_ref[...][:, 0:c]
    h_ref[...] = h
    ef_ref[...] = jnp.concatenate([ef, jnp.zeros((ef.shape[0], 1), jnp.float32)],
                                  axis=1)
    _accum_stats(s_ref, h, pl.program_id(0))


def _lnp1_body(g_ref, ef_ref, w_ref, p_ref, h_ref, s_ref, *, cin):
    x = jnp.concatenate([g_ref[...][:, 0:cin], ef_ref[...][:, 0:3]], axis=1)
    c = w_ref.shape[1]
    h = _bdot(x, w_ref[...]) + p_ref[...][:, 0:c]
    h_ref[...] = h
    _accum_stats(s_ref, h, pl.program_id(0))


def _mid_body(hin_ref, s_in_ref, pprev_ref, w_ref, p_ref, h_ref, s_ref, *, c):
    h = _norm_lrelu(hin_ref[...], s_in_ref[...], pprev_ref[...], c)
    h = _bdot(h, w_ref[...]) + p_ref[...][:, 0:c]
    h_ref[...] = h
    _accum_stats(s_ref, h, pl.program_id(0))


def _pool_body(h0_ref, h1_ref, h2_ref, h3_ref, s_ref, p_ref, out_ref, *, c):
    s = s_ref[...]
    p = p_ref[...]
    a = jnp.maximum(_norm_lrelu(h0_ref[...], s, p, c),
                    _norm_lrelu(h1_ref[...], s, p, c))
    b = jnp.maximum(_norm_lrelu(h2_ref[...], s, p, c),
                    _norm_lrelu(h3_ref[...], s, p, c))
    m = jnp.maximum(a, b)
    out_ref[...] = jnp.concatenate(
        [m, jnp.zeros((m.shape[0], _GW - c), jnp.float32)], axis=1)


def _final_body(h0_ref, h1_ref, h2_ref, h3_ref, s_ref, p_ref, fcw_ref,
                fcb_ref, flow_ref, out_ref):
    c = 64
    s = s_ref[...]
    p = p_ref[...]
    a = jnp.maximum(_norm_lrelu(h0_ref[...], s, p, c),
                    _norm_lrelu(h1_ref[...], s, p, c))
    b = jnp.maximum(_norm_lrelu(h2_ref[...], s, p, c),
                    _norm_lrelu(h3_ref[...], s, p, c))
    m = jnp.maximum(a, b)
    y = _bdot(m, fcw_ref[...]) + fcb_ref[...]
    out_ref[...] = flow_ref[...] + y


def _pack_layer_params(p, w):
    def pack(i):
        return jnp.concatenate(
            [p['fc%d_b' % i], p['bn%d_g' % i], p['bn%d_b' % i]]
        ).reshape(1, 3 * w)
    return (p['fc1_w'].T, pack(1), p['fc2_w'].T, pack(2), p['fc3_w'].T, pack(3))


def _edge_specs(c):
    blocks_per_n = _N // _BN
    return [
        pl.BlockSpec((_BN, c), functools.partial(
            lambda k, i: (k * blocks_per_n + i, 0), k))
        for k in range(_K)
    ]


def _stats_spec(c):
    return pl.BlockSpec((8, c), lambda i: (0, 0))


def _full(shape):
    return pl.BlockSpec(shape, lambda i: tuple(0 for _ in shape))


def _setconv(gathered, ef, pc2d, params, cin, c, first):
    wt1, p1, wt2, p2, wt3, p3 = _pack_layer_params(params, c)
    grid_e = (_E // _BE,)
    h_shape = jax.ShapeDtypeStruct((_E, c), jnp.float32)
    s_shape = jax.ShapeDtypeStruct((8, c), jnp.float32)

    if first:
        h1, ef, s1 = pl.pallas_call(
            _l1p1_body,
            grid=grid_e,
            in_specs=[
                pl.BlockSpec((_BE, _GW), lambda i: (i, 0)),
                pl.BlockSpec((_BE, 3),
                             lambda i: (i % (_N // _BE), 0)),
                _full(wt1.shape), _full(p1.shape),
            ],
            out_specs=[pl.BlockSpec((_BE, c), lambda i: (i, 0)),
                       pl.BlockSpec((_BE, 4), lambda i: (i, 0)),
                       _stats_spec(c)],
            out_shape=(h_shape, jax.ShapeDtypeStruct((_E, 4), jnp.float32),
                       s_shape),
        )(gathered, pc2d, wt1, p1)
    else:
        h1, s1 = pl.pallas_call(
            functools.partial(_lnp1_body, cin=cin),
            grid=grid_e,
            in_specs=[
                pl.BlockSpec((_BE, _GW), lambda i: (i, 0)),
                pl.BlockSpec((_BE, 4), lambda i: (i, 0)),
                _full(wt1.shape), _full(p1.shape),
            ],
            out_specs=[pl.BlockSpec((_BE, c), lambda i: (i, 0)),
                       _stats_spec(c)],
            out_shape=(h_shape, s_shape),
        )(gathered, ef, wt1, p1)

    def mid(hin, s_in, pprev, wt, p):
        return pl.pallas_call(
            functools.partial(_mid_body, c=c),
            grid=grid_e,
            in_specs=[
                pl.BlockSpec((_BE, c), lambda i: (i, 0)),
                _stats_spec(c), _full(pprev.shape),
                _full(wt.shape), _full(p.shape),
            ],
            out_specs=[pl.BlockSpec((_BE, c), lambda i: (i, 0)),
                       _stats_spec(c)],
            out_shape=(h_shape, s_shape),
        )(hin, s_in, pprev, wt, p)

    h2, s2 = mid(h1, s1, p1, wt2, p2)
    h3, s3 = mid(h2, s2, p2, wt3, p3)
    return h3, s3, p3, ef


def kernel(pc, flow, params):
    pc2d = pc.reshape(_N, 3)
    flow2d = flow.reshape(_N, 3)

    idx = _knn(pc2d)
    edges = idx.T.reshape(_E)

    grid_n = (_N // _BN,)

    table0 = jnp.concatenate(
        [flow2d, pc2d, jnp.zeros((_N, _GW - 6), jnp.float32)], axis=1)
    g0 = _sc_gather(table0, edges)
    h3, s3, p3, ef = _setconv(g0, None, pc2d, params['sc1'], 3, 16, True)
    x1 = pl.pallas_call(
        functools.partial(_pool_body, c=16),
        grid=grid_n,
        in_specs=_edge_specs(16) + [_stats_spec(16), _full(p3.shape)],
        out_specs=pl.BlockSpec((_BN, _GW), lambda i: (i, 0)),
        out_shape=jax.ShapeDtypeStruct((_N, _GW), jnp.float32),
    )(h3, h3, h3, h3, s3, p3)

    g1 = _sc_gather(x1, edges)
    h3, s3, p3, _ = _setconv(g1, ef, pc2d, params['sc2'], 16, 32, False)
    x2 = pl.pallas_call(
        functools.partial(_pool_body, c=32),
        grid=grid_n,
        in_specs=_edge_specs(32) + [_stats_spec(32), _full(p3.shape)],
        out_specs=pl.BlockSpec((_BN, _GW), lambda i: (i, 0)),
        out_shape=jax.ShapeDtypeStruct((_N, _GW), jnp.float32),
    )(h3, h3, h3, h3, s3, p3)

    g2 = _sc_gather(x2, edges)
    h3, s3, p3, _ = _setconv(g2, ef, pc2d, params['sc3'], 32, 64, False)
    fcw = params['fc_w'].T
    fcb = params['fc_b'].reshape(1, 3)
    out = pl.pallas_call(
        _final_body,
        grid=grid_n,
        in_specs=_edge_specs(64) + [
            _stats_spec(64), _full(p3.shape), _full(fcw.shape),
            _full(fcb.shape),
            pl.BlockSpec((_BN, 3), lambda i: (i, 0)),
        ],
        out_specs=pl.BlockSpec((_BN, 3), lambda i: (i, 0)),
        out_shape=jax.ShapeDtypeStruct((_N, 3), jnp.float32),
    )(h3, h3, h3, h3, s3, p3, fcw, fcb, flow2d)

    return out.reshape(1, _N, 3)


# speedup vs baseline: 1.1764x; 1.1341x over previous
import functools

import jax
import jax.numpy as jnp
from jax import lax
from jax.experimental import pallas as pl
from jax.experimental.pallas import tpu as pltpu
from jax.experimental.pallas import tpu_sc as plsc

_N = 8192
_K = 4
_E = _N * _K

_SC_CORES = 2
_SC_SUBCORES = 16
_SC_WORKERS = _SC_CORES * _SC_SUBCORES

_KNN_ROWS = 512
_BE = 4096
_BN = 4096
_GW = 128
_GCHUNK = 512


def _knn_body(pc_blk_ref, pcT_ref, idx_ref):
    pcb = pc_blk_ref[...]
    pcT = pcT_ref[...]
    r = pcb.shape[0]
    sq_c = jnp.sum(pcT * pcT, axis=0, keepdims=True)
    sq_r = jnp.sum(pcb * pcb, axis=1, keepdims=True)
    dot = jnp.dot(pcb.astype(jnp.bfloat16), pcT.astype(jnp.bfloat16),
                  preferred_element_type=jnp.float32)
    d = sq_r + sq_c - 2.0 * dot
    lane = lax.broadcasted_iota(jnp.int32, (r, 128), 1)
    big_i = jnp.int32(2**30)
    inf = jnp.float32(jnp.inf)
    v0 = d[:, 0:128]
    i0 = jnp.zeros((r, 128), jnp.int32)
    v1 = jnp.full((r, 128), inf)
    i1 = jnp.zeros((r, 128), jnp.int32)
    v2 = jnp.full((r, 128), inf)
    i2 = jnp.zeros((r, 128), jnp.int32)
    for c in range(1, _N // 128):
        dc = d[:, c * 128:(c + 1) * 128]
        cc = jnp.int32(c)
        c0 = dc < v0
        c1 = dc < v1
        c2 = dc < v2
        v2 = jnp.where(c1, v1, jnp.where(c2, dc, v2))
        i2 = jnp.where(c1, i1, jnp.where(c2, cc, i2))
        v1 = jnp.where(c0, v0, jnp.where(c1, dc, v1))
        i1 = jnp.where(c0, i0, jnp.where(c1, cc, i1))
        v0 = jnp.where(c0, dc, v0)
        i0 = jnp.where(c0, cc, i0)
    w = jnp.concatenate([v0, v1, v2], axis=1)
    jf = jnp.concatenate(
        [i0 * 128 + lane, i1 * 128 + lane, i2 * 128 + lane], axis=1)
    cols = []
    for t in range(_K):
        m = jnp.min(w, axis=1, keepdims=True)
        j = jnp.min(jnp.where(w == m, jf, big_i), axis=1, keepdims=True)
        cols.append(j)
        if t < _K - 1:
            w = jnp.where(jf == j, inf, w)
    idx_ref[...] = jnp.concatenate(cols, axis=1)


def _knn(pc2d):
    pcT = pc2d.T
    return pl.pallas_call(
        _knn_body,
        grid=(_N // _KNN_ROWS,),
        in_specs=[
            pl.BlockSpec((_KNN_ROWS, 3), lambda i: (i, 0)),
            pl.BlockSpec((3, _N), lambda i: (0, 0)),
        ],
        out_specs=pl.BlockSpec((_KNN_ROWS, _K), lambda i: (i, 0)),
        out_shape=jax.ShapeDtypeStruct((_N, _K), jnp.int32),
    )(pc2d, pcT)


def _sc_gather(table, idx):
    b_per_w = _E // _SC_WORKERS
    n_chunks = b_per_w // _GCHUNK
    mesh = plsc.VectorSubcoreMesh(core_axis_name="c", subcore_axis_name="s")

    @functools.partial(
        pl.kernel,
        mesh=mesh,
        out_type=jax.ShapeDtypeStruct((_E, _GW), jnp.float32),
        scratch_types=[
            pltpu.VMEM((b_per_w,), jnp.int32),
            pltpu.VMEM((_GCHUNK, _GW), jnp.float32),
            pltpu.SemaphoreType.DMA,
        ],
    )
    def k(table_hbm, idx_hbm, out_hbm, idx_v, rows_v, sem):
        wid = lax.axis_index("s") * _SC_CORES + lax.axis_index("c")
        base = wid * b_per_w
        pltpu.sync_copy(idx_hbm.at[pl.ds(base, b_per_w)], idx_v)
        for c in range(n_chunks):
            pltpu.async_copy(
                table_hbm.at[idx_v.at[pl.ds(c * _GCHUNK, _GCHUNK)]],
                rows_v, sem).wait()
            pltpu.sync_copy(rows_v,
                            out_hbm.at[pl.ds(base + c * _GCHUNK, _GCHUNK)])

    return k(table, idx)


_PB = _E // _BE
_BN2 = _N // _PB


def _bdot(x, w):
    return jnp.dot(x.astype(jnp.bfloat16), w.astype(jnp.bfloat16),
                   preferred_element_type=jnp.float32)


def _norm_lrelu(h, s, pack, c):
    mean = s[0:1, :] * (1.0 / _E)
    var = s[1:2, :] * (1.0 / _E) - mean * mean
    g = pack[:, c:2 * c]
    be = pack[:, 2 * c:3 * c]
    h = (h - mean) / jnp.sqrt(var + 1e-5) * g + be
    return jnp.where(h >= 0, h, 0.1 * h)


def _accum(s_ref, h, b):
    c = h.shape[1]
    part = jnp.concatenate(
        [jnp.sum(h, axis=0, keepdims=True),
         jnp.sum(h * h, axis=0, keepdims=True),
         jnp.zeros((6, c), jnp.float32)], axis=0)

    @pl.when(b == 0)
    def _():
        s_ref[...] = part

    @pl.when(b != 0)
    def _():
        s_ref[...] = s_ref[...] + part


def _layer_kernel_body(g_ref, aux_ref, flow_ref, w1, p1, w2, p2, w3, p3,
                       fcw_ref, fcb_ref, out_ref, ef_ref, h_scr, s0, s1, s2,
                       *, c, cin, first, last):
    p = pl.program_id(0)
    b = pl.program_id(1)

    @pl.when(p == 0)
    def _():
        g = g_ref[...]
        if first:
            ef = g[:, 3:6] - aux_ref[...][:, 0:3]
            x = jnp.concatenate([g[:, 0:3], ef], axis=1)
            ef_ref[...] = jnp.concatenate(
                [ef, jnp.zeros((_BE, 1), jnp.float32)], axis=1)
        else:
            x = jnp.concatenate([g[:, 0:cin], aux_ref[...][:, 0:3]], axis=1)
        h = _bdot(x, w1[...]) + p1[...][:, 0:c]
        h_scr[pl.ds(b * _BE, _BE), :] = h
        _accum(s0, h, b)

    @pl.when(p == 1)
    def _():
        h = _norm_lrelu(h_scr[pl.ds(b * _BE, _BE), :], s0[...], p1[...], c)
        h = _bdot(h, w2[...]) + p2[...][:, 0:c]
        h_scr[pl.ds(b * _BE, _BE), :] = h
        _accum(s1, h, b)

    @pl.when(p == 2)
    def _():
        h = _norm_lrelu(h_scr[pl.ds(b * _BE, _BE), :], s1[...], p2[...], c)
        h = _bdot(h, w3[...]) + p3[...][:, 0:c]
        h_scr[pl.ds(b * _BE, _BE), :] = h
        _accum(s2, h, b)

    @pl.when(p == 3)
    def _():
        s = s2[...]
        pk = p3[...]
        hs = [_norm_lrelu(h_scr[pl.ds(k * _N + b * _BN2, _BN2), :], s, pk, c)
              for k in range(_K)]
        m = jnp.maximum(jnp.maximum(hs[0], hs[1]), jnp.maximum(hs[2], hs[3]))
        if last:
            y = _bdot(m, fcw_ref[...]) + fcb_ref[...]
            out_ref[...] = flow_ref[...] + y
        else:
            out_ref[...] = jnp.concatenate(
                [m, jnp.zeros((_BN2, _GW - c), jnp.float32)], axis=1)


def _pack_layer_params(pp, w):
    def pack(i):
        return jnp.concatenate(
            [pp['fc%d_b' % i], pp['bn%d_g' % i], pp['bn%d_b' % i]]
        ).reshape(1, 3 * w)
    return (pp['fc1_w'].T, pack(1), pp['fc2_w'].T, pack(2),
            pp['fc3_w'].T, pack(3))


def _setconv_layer(gathered, aux, flow2d, fcw, fcb, pp, cin, c, first, last):
    wt1, pk1, wt2, pk2, wt3, pk3 = _pack_layer_params(pp, c)

    def g_map(p, b):
        return (jnp.where(p == 0, b, _PB - 1), 0)

    def aux_map(p, b):
        nb = _PB - 1 if not first else 1
        sel = b if not first else b % (_N // _BE)
        return (jnp.where(p == 0, sel, nb), 0)

    def out_map(p, b):
        return (jnp.where(p == 3, b, 0), 0)

    in_specs = [
        pl.BlockSpec((_BE, _GW), g_map),
        pl.BlockSpec((_BE, 4) if not first else (_BE, 3), aux_map),
        pl.BlockSpec((_BN2, 3), out_map),
        pl.BlockSpec(wt1.shape, lambda p, b: (0, 0)),
        pl.BlockSpec(pk1.shape, lambda p, b: (0, 0)),
        pl.BlockSpec(wt2.shape, lambda p, b: (0, 0)),
        pl.BlockSpec(pk2.shape, lambda p, b: (0, 0)),
        pl.BlockSpec(wt3.shape, lambda p, b: (0, 0)),
        pl.BlockSpec(pk3.shape, lambda p, b: (0, 0)),
        pl.BlockSpec(fcw.shape, lambda p, b: (0, 0)),
        pl.BlockSpec(fcb.shape, lambda p, b: (0, 0)),
    ]
    if last:
        out_shape = [jax.ShapeDtypeStruct((_N, 3), jnp.float32)]
        out_specs = [pl.BlockSpec((_BN2, 3), out_map)]
    else:
        out_shape = [jax.ShapeDtypeStruct((_N, _GW), jnp.float32)]
        out_specs = [pl.BlockSpec((_BN2, _GW), out_map)]
    out_shape.append(jax.ShapeDtypeStruct((_E, 4), jnp.float32))
    out_specs.append(pl.BlockSpec((_BE, 4), g_map))

    res = pl.pallas_call(
        functools.partial(_layer_kernel_body, c=c, cin=cin,
                          first=first, last=last),
        grid=(4, _PB),
        in_specs=in_specs,
        out_specs=out_specs,
        out_shape=tuple(out_shape),
        scratch_shapes=[
            pltpu.VMEM((_E, c), jnp.float32),
            pltpu.VMEM((8, c), jnp.float32),
            pltpu.VMEM((8, c), jnp.float32),
            pltpu.VMEM((8, c), jnp.float32),
        ],
    )(gathered, aux, flow2d, wt1, pk1, wt2, pk2, wt3, pk3, fcw, fcb)
    return res


def kernel(pc, flow, params):
    pc2d = pc.reshape(_N, 3)
    flow2d = flow.reshape(_N, 3)

    idx = _knn(pc2d)
    edges = idx.T.reshape(_E)

    fcw = params['fc_w'].T
    fcb = params['fc_b'].reshape(1, 3)

    table0 = jnp.concatenate(
        [flow2d, pc2d, jnp.zeros((_N, _GW - 6), jnp.float32)], axis=1)
    g0 = _sc_gather(table0, edges)
    x1, ef = _setconv_layer(g0, pc2d, flow2d, fcw, fcb, params['sc1'],
                            3, 16, True, False)

    g1 = _sc_gather(x1, edges)
    x2, _ = _setconv_layer(g1, ef, flow2d, fcw, fcb, params['sc2'],
                           16, 32, False, False)

    g2 = _sc_gather(x2, edges)
    out, _ = _setconv_layer(g2, ef, flow2d, fcw, fcb, params['sc3'],
                            32, 64, False, True)

    return out.reshape(1, _N, 3)
